# Initial kernel scaffold; baseline (speedup 1.0000x reference)
import jax
import jax.numpy as jnp
from jax.experimental import pallas as pl


def kernel(token_ids, pos_table):
    raise NotImplementedError("write your pallas kernel here")


# speedup vs baseline: 2.5158x; 2.5158x over previous
import jax
import jax.numpy as jnp
from jax.experimental import pallas as pl


def _copy_body(in_ref, out_ref):
    out_ref[...] = in_ref[...]


def kernel(token_ids, pos_table):
    seq_len = token_ids.shape[-1]
    embed_dim = pos_table.shape[1]
    block = 512
    while seq_len % block:
        block //= 2
    return pl.pallas_call(
        _copy_body,
        grid=(seq_len // block,),
        in_specs=[pl.BlockSpec((block, embed_dim), lambda i: (i, 0))],
        out_specs=pl.BlockSpec((block, embed_dim), lambda i: (i, 0)),
        out_shape=jax.ShapeDtypeStruct((seq_len, embed_dim), pos_table.dtype),
    )(pos_table)
